# baseline (speedup 1.0000x reference)
import functools
import math

import jax
import jax.numpy as jnp
from jax import lax
from jax.experimental import pallas as pl
from jax.experimental.pallas import tpu as pltpu
from jax.experimental.pallas import tpu_sc as plsc

_V = 100000
_B = 1024
_SMOOTH = 0.1 / (_V - 2)
_CONF = 0.9
_ENT = (_V - 2) * _SMOOTH * math.log(_SMOOTH) + _CONF * math.log(_CONF)
_BLK = 2048
_GRID = (_V + _BLK - 1) // _BLK

_NC, _NS, _L = 2, 16, 16
_NW = _NC * _NS
_BPW = _B // _NW


def _tc_body(tgt_ref, out_ref, acc_ref):
    j = pl.program_id(0)
    d = out_ref[...]
    m = (tgt_ref[...] != 0).astype(jnp.float32)

    @pl.when(j == 0)
    def _():
        acc_ref[...] = jnp.zeros((1, 1), jnp.float32)

    @pl.when(j < _GRID - 1)
    def _():
        acc_ref[...] += jnp.sum(m * d)

    @pl.when(j == _GRID - 1)
    def _():
        col = j * _BLK + jax.lax.broadcasted_iota(jnp.int32, (_B, _BLK), 1)
        acc_ref[...] += jnp.sum(jnp.where(col < _V, m * d, 0.0))


def _tc_sum(output, t2):
    acc = pl.pallas_call(
        _tc_body,
        grid=(_GRID,),
        in_specs=[
            pl.BlockSpec((_B, 1), lambda j: (0, 0)),
            pl.BlockSpec((_B, _BLK), lambda j: (0, j)),
        ],
        out_specs=pl.BlockSpec((1, 1), lambda j: (0, 0)),
        out_shape=jax.ShapeDtypeStruct((1, 1), jnp.float32),
    )(t2, output)
    return acc[0, 0]


def _sc_gather_body(flat_hbm, idxt_hbm, idx0_hbm, tgt_hbm, out_hbm,
                    idxt_v, idx0_v, tgt_v, g_v, z_v, st_v, sem):
    wid = lax.axis_index("s") * _NC + lax.axis_index("c")
    base = wid * _BPW
    pltpu.sync_copy(idxt_hbm.at[pl.ds(base, _BPW)], idxt_v)
    pltpu.sync_copy(idx0_hbm.at[pl.ds(base, _BPW)], idx0_v)
    pltpu.sync_copy(tgt_hbm.at[pl.ds(base, _BPW)], tgt_v)
    cp1 = pltpu.async_copy(flat_hbm.at[idxt_v], g_v, sem)
    cp2 = pltpu.async_copy(flat_hbm.at[idx0_v], z_v, sem)
    cp1.wait()
    cp2.wait()

    accg = jnp.zeros((_L,), jnp.float32)
    accz = jnp.zeros((_L,), jnp.float32)
    accn = jnp.zeros((_L,), jnp.float32)
    for k in range(_BPW // _L):
        tv = tgt_v[pl.ds(k * _L, _L)]
        valid = tv != 0
        accg += jnp.where(valid, g_v[pl.ds(k * _L, _L)], 0.0)
        accz += jnp.where(valid, z_v[pl.ds(k * _L, _L)], 0.0)
        accn += jnp.where(valid, 1.0, 0.0)
    st_v[0, :] = accg
    st_v[1, :] = accz
    st_v[2, :] = accn
    pltpu.sync_copy(st_v, out_hbm.at[wid])


@functools.cache
def _make_sc_gather():
    return pl.kernel(
        _sc_gather_body,
        mesh=plsc.VectorSubcoreMesh(core_axis_name="c", subcore_axis_name="s"),
        out_type=jax.ShapeDtypeStruct((_NW, 3, _L), jnp.float32),
        scratch_types=[
            pltpu.VMEM((_BPW,), jnp.int32),
            pltpu.VMEM((_BPW,), jnp.int32),
            pltpu.VMEM((_BPW,), jnp.int32),
            pltpu.VMEM((_BPW,), jnp.float32),
            pltpu.VMEM((_BPW,), jnp.float32),
            pltpu.VMEM((3, _L), jnp.float32),
            pltpu.SemaphoreType.DMA,
        ],
    )


def kernel(output, target):
    t2 = target.reshape(_B, 1)
    rows = jnp.arange(_B, dtype=jnp.int32) * _V
    idx_t = rows + target
    parts = _make_sc_gather()(output.reshape(-1), idx_t, rows, target)
    s_all = _tc_sum(output, t2)
    p = jnp.sum(parts, axis=(0, 2))
    s_tgt, s_col0, n_valid = p[0], p[1], p[2]
    return (_ENT * n_valid - _SMOOTH * s_all + _SMOOTH * s_col0
            - (_CONF - _SMOOTH) * s_tgt)


# speedup vs baseline: 1.9060x; 1.9060x over previous
import math

import jax
import jax.numpy as jnp
from jax.experimental import pallas as pl
from jax.experimental.pallas import tpu as pltpu

_V = 100000
_B = 1024
_SMOOTH = 0.1 / (_V - 2)
_CONF = 0.9
_ENT = (_V - 2) * _SMOOTH * math.log(_SMOOTH) + _CONF * math.log(_CONF)
_BLK = 2048
_GRID = (_V + _BLK - 1) // _BLK
_NCH = _BLK // 128


def _body(tgt_ref, out_ref, loss_ref, racc_ref, tacc_ref, c0_ref):
    j = pl.program_id(0)
    d = out_ref[...]
    t = tgt_ref[...]
    lane = jax.lax.broadcasted_iota(jnp.int32, (_B, 128), 1)

    @pl.when(j == 0)
    def _():
        m0 = (t != 0).astype(jnp.float32)
        c0_ref[...] = jnp.sum(m0 * d[:, 0:1]).reshape(1, 1)
        racc_ref[...] = jnp.zeros((_B, 128), jnp.float32)
        tacc_ref[...] = jnp.zeros((_B, 128), jnp.float32)

    @pl.when(j < _GRID - 1)
    def _():
        racc = racc_ref[...]
        tacc = tacc_ref[...]
        for k in range(_NCH):
            dk = d[:, k * 128:(k + 1) * 128]
            racc += dk
            tacc += jnp.where(lane == t - (j * _BLK + k * 128), dk, 0.0)
        racc_ref[...] = racc
        tacc_ref[...] = tacc

    @pl.when(j == _GRID - 1)
    def _():
        racc = racc_ref[...]
        tacc = tacc_ref[...]
        for k in range(_NCH):
            base = j * _BLK + k * 128
            dk = jnp.where(lane + base < _V, d[:, k * 128:(k + 1) * 128], 0.0)
            racc += dk
            tacc += jnp.where(lane == t - base, dk, 0.0)
        m = (t != 0).astype(jnp.float32)
        n_valid = jnp.sum(m)
        s_all = jnp.sum(m * racc)
        s_tgt = jnp.sum(m * tacc)
        loss_ref[...] = (_ENT * n_valid - _SMOOTH * s_all
                        + _SMOOTH * c0_ref[...]
                        - (_CONF - _SMOOTH) * s_tgt)


def kernel(output, target):
    t2 = target.reshape(_B, 1)
    acc = pl.pallas_call(
        _body,
        grid=(_GRID,),
        in_specs=[
            pl.BlockSpec((_B, 1), lambda j: (0, 0)),
            pl.BlockSpec((_B, _BLK), lambda j: (0, j)),
        ],
        out_specs=pl.BlockSpec((1, 1), lambda j: (0, 0)),
        out_shape=jax.ShapeDtypeStruct((1, 1), jnp.float32),
        scratch_shapes=[
            pltpu.VMEM((_B, 128), jnp.float32),
            pltpu.VMEM((_B, 128), jnp.float32),
            pltpu.VMEM((1, 1), jnp.float32),
        ],
    )(t2, output)
    return acc[0, 0]


# speedup vs baseline: 2.0035x; 1.0511x over previous
import math

import jax
import jax.numpy as jnp
from jax.experimental import pallas as pl
from jax.experimental.pallas import tpu as pltpu

_V = 100000
_B = 1024
_SMOOTH = 0.1 / (_V - 2)
_CONF = 0.9
_ENT = (_V - 2) * _SMOOTH * math.log(_SMOOTH) + _CONF * math.log(_CONF)
_BLK = 2048
_GRID = (_V + _BLK - 1) // _BLK
_NCH = _BLK // 128


def _body(tgt_ref, out_ref, loss_ref, sacc_ref, tacc_ref, c0_ref):
    j = pl.program_id(0)
    d = out_ref[...]
    t = tgt_ref[...]
    m = (t != 0).astype(jnp.float32)
    lane = jax.lax.broadcasted_iota(jnp.int32, (_B, 128), 1)

    @pl.when(j == 0)
    def _():
        c0_ref[...] = jnp.sum(m * d[:, 0:1]).reshape(1, 1)
        sacc_ref[...] = jnp.zeros((1, 1), jnp.float32)
        tacc_ref[...] = jnp.zeros((1, 1), jnp.float32)

    def block_partials(masked_tail):
        psum = jnp.zeros((_B, 128), jnp.float32)
        tsum = jnp.zeros((_B, 128), jnp.float32)
        for k in range(_NCH):
            base = j * _BLK + k * 128
            dk = d[:, k * 128:(k + 1) * 128]
            if masked_tail:
                dk = jnp.where(lane + base < _V, dk, 0.0)
            psum += dk
            tsum += jnp.where(lane == t - base, dk, 0.0)
        return psum, tsum

    @pl.when(j < _GRID - 1)
    def _():
        psum, tsum = block_partials(False)
        sacc_ref[...] += jnp.sum(m * psum)
        tacc_ref[...] += jnp.sum(m * tsum)

    @pl.when(j == _GRID - 1)
    def _():
        psum, tsum = block_partials(True)
        s_all = sacc_ref[...] + jnp.sum(m * psum)
        s_tgt = tacc_ref[...] + jnp.sum(m * tsum)
        n_valid = jnp.sum(m)
        loss_ref[...] = (_ENT * n_valid - _SMOOTH * s_all
                        + _SMOOTH * c0_ref[...]
                        - (_CONF - _SMOOTH) * s_tgt)


def kernel(output, target):
    t2 = target.reshape(_B, 1)
    acc = pl.pallas_call(
        _body,
        grid=(_GRID,),
        in_specs=[
            pl.BlockSpec((_B, 1), lambda j: (0, 0)),
            pl.BlockSpec((_B, _BLK), lambda j: (0, j)),
        ],
        out_specs=pl.BlockSpec((1, 1), lambda j: (0, 0)),
        out_shape=jax.ShapeDtypeStruct((1, 1), jnp.float32),
        scratch_shapes=[
            pltpu.VMEM((1, 1), jnp.float32),
            pltpu.VMEM((1, 1), jnp.float32),
            pltpu.VMEM((1, 1), jnp.float32),
        ],
    )(t2, output)
    return acc[0, 0]
